# baseline (speedup 1.0000x reference)
import functools

import jax
import jax.numpy as jnp
from jax import lax
from jax.experimental import pallas as pl
from jax.experimental.pallas import tpu as pltpu
from jax.experimental.pallas import tpu_sc as plsc

NB, T = 4, 8192
D = 1024
NC, NS = 2, 16
NW = NC * NS
B_PER_W = NB * T // NW
WPB = T // B_PER_W
CHUNK = 32
NCHUNK = B_PER_W // CHUNK


def _gather_kernel(table_hbm, idx_hbm, out_hbm, idx_v, buf0, buf1, sem0, sem1):
    wid = lax.axis_index("s") * NC + lax.axis_index("c")
    b = wid // WPB
    t0 = (wid % WPB) * B_PER_W
    pltpu.sync_copy(idx_hbm.at[b, pl.ds(t0, B_PER_W)], idx_v)

    def gather_cp(g, buf, sem):
        return pltpu.make_async_copy(
            table_hbm.at[idx_v.at[pl.ds(g * CHUNK, CHUNK)]], buf, sem
        )

    def write(g, buf):
        pltpu.sync_copy(buf, out_hbm.at[b, pl.ds(t0 + g * CHUNK, CHUNK)])

    gather_cp(0, buf0, sem0).start()

    @pl.loop(0, NCHUNK, step=2)
    def _(g):
        gather_cp(g + 1, buf1, sem1).start()
        gather_cp(g, buf0, sem0).wait()
        write(g, buf0)

        @pl.when(g + 2 < NCHUNK)
        def _():
            gather_cp(g + 2, buf0, sem0).start()

        gather_cp(g + 1, buf1, sem1).wait()
        write(g + 1, buf1)


def kernel(position_ids, embedding_weight):
    mesh = plsc.VectorSubcoreMesh(core_axis_name="c", subcore_axis_name="s")
    k = functools.partial(
        pl.kernel,
        mesh=mesh,
        out_type=jax.ShapeDtypeStruct((NB, T, D), jnp.float32),
        scratch_types=[
            pltpu.VMEM((B_PER_W,), jnp.int32),
            pltpu.VMEM((CHUNK, D), jnp.float32),
            pltpu.VMEM((CHUNK, D), jnp.float32),
            pltpu.SemaphoreType.DMA,
            pltpu.SemaphoreType.DMA,
        ],
    )(_gather_kernel)
    return k(embedding_weight, position_ids)
